# Initial kernel scaffold; baseline (speedup 1.0000x reference)
import jax
import jax.numpy as jnp
from jax.experimental import pallas as pl


def kernel(x, topk_index, w0, w1):
    raise NotImplementedError("write your pallas kernel here")


# speedup vs baseline: 8.1757x; 8.1757x over previous
import functools

import jax
import jax.numpy as jnp
from jax.experimental import pallas as pl
from jax.experimental.pallas import tpu as pltpu

EN = 16
KN = 2
DM = 768
DF = 3072
TN = 2048
RB = 256
RP = TN * KN + EN * RB
NBLK = RP // RB


def _gelu_exact(v):
    return 0.5 * v * (1.0 + jax.lax.erf(v * 0.7071067811865476))


def _gemm_body(be_ref, x_ref, w0_ref, w1_ref, o_ref):
    xb = x_ref[...]
    h = _gelu_exact(jnp.dot(xb, w0_ref[0], preferred_element_type=jnp.float32))
    o_ref[...] = jnp.dot(h, w1_ref[0], preferred_element_type=jnp.float32)


def _grouped_gemm(block_expert, rep_x, w0, w1, interpret=False):
    return pl.pallas_call(
        _gemm_body,
        grid_spec=pltpu.PrefetchScalarGridSpec(
            num_scalar_prefetch=1,
            grid=(NBLK,),
            in_specs=[
                pl.BlockSpec((RB, DM), lambda j, be: (j, 0)),
                pl.BlockSpec((1, DM, DF), lambda j, be: (be[j], 0, 0)),
                pl.BlockSpec((1, DF, DM), lambda j, be: (be[j], 0, 0)),
            ],
            out_specs=pl.BlockSpec((RB, DM), lambda j, be: (j, 0)),
        ),
        out_shape=jax.ShapeDtypeStruct((RP, DM), jnp.float32),
        interpret=interpret,
    )(block_expert, rep_x, w0, w1)


def kernel(x, topk_index, w0, w1):
    e = topk_index.reshape(-1)
    oh = (e[:, None] == jnp.arange(EN, dtype=e.dtype)).astype(jnp.int32)
    cs = jnp.cumsum(oh, axis=0)
    rank = jnp.sum((cs - oh) * oh, axis=1)
    counts = cs[-1]
    padded = ((counts + RB - 1) // RB) * RB
    base = jnp.concatenate(
        [jnp.zeros((1,), jnp.int32), jnp.cumsum(padded)[:-1].astype(jnp.int32)]
    )
    pos = rank + jnp.sum(oh * base[None, :], axis=1)
    blk_base = base // RB
    jidx = jnp.arange(NBLK, dtype=jnp.int32)
    block_expert = (
        jnp.sum((blk_base[None, :] <= jidx[:, None]).astype(jnp.int32), axis=1) - 1
    ).astype(jnp.int32)
    src = jnp.zeros((RP,), jnp.int32).at[pos].set(
        jnp.arange(TN * KN, dtype=jnp.int32) // KN
    )
    rep_x = jnp.take(x, src, axis=0)
    y = _grouped_gemm(block_expert, rep_x, w0, w1)
    outf = jnp.take(y, pos, axis=0)
    return outf.reshape(TN, KN, DM).sum(axis=1)
